# baseline (speedup 1.0000x reference)
import jax
import jax.numpy as jnp
from jax.experimental import pallas as pl
from jax.experimental.pallas import tpu as pltpu

S, D, H, E, TOPK, F = 2048, 1024, 64, 16, 2, 64

CH_OFF = (0, 768, 1536, 1792)
CH_LEN = (768, 768, 256, 256)
NC = len(CH_OFF)
EB = 1024
CS = None


def _moe_kernel(x_hbm, x2_hbm, wout_hbm, wgin_t_hbm, wglin_t_hbm, wexp_t_hbm,
                out_hbm, stash_ref, pad_ref, wcat_ref, wout_ref, wgin_ref,
                wglin_ref, yt_ref, zbuf_ref, xsem, wsem, esem, osem):
    for k in range(NC):
        src_ref = x_hbm if k % 2 == 0 else x2_hbm
        pltpu.make_async_copy(
            src_ref.at[pl.ds(CH_OFF[k], CH_LEN[k]), :],
            stash_ref.at[pl.ds(CH_OFF[k], CH_LEN[k]), :], xsem.at[k]).start()
    pltpu.make_async_copy(wout_hbm, wout_ref, wsem.at[0]).start()
    pltpu.make_async_copy(wgin_t_hbm, wgin_ref, wsem.at[1]).start()
    pltpu.make_async_copy(wglin_t_hbm, wglin_ref, wsem.at[2]).start()

    with jax.named_scope("stream_gate"):
        pltpu.make_async_copy(wout_hbm, wout_ref, wsem.at[0]).wait()
        v = jnp.zeros((1, D), dtype=jnp.float32)
        for k in range(NC):
            pltpu.make_async_copy(
                x_hbm.at[pl.ds(CH_OFF[k], CH_LEN[k]), :],
                stash_ref.at[pl.ds(CH_OFF[k], CH_LEN[k]), :], xsem.at[k]).wait()
            v = v + jnp.dot(wout_ref[:, CH_OFF[k]:CH_OFF[k] + CH_LEN[k]],
                            stash_ref[CH_OFF[k]:CH_OFF[k] + CH_LEN[k], :],
                            preferred_element_type=jnp.float32)

    pltpu.make_async_copy(wgin_t_hbm, wgin_ref, wsem.at[1]).wait()
    pltpu.make_async_copy(wglin_t_hbm, wglin_ref, wsem.at[2]).wait()
    u = jax.lax.dot_general(v, wgin_ref[...],
                            dimension_numbers=(((1,), (1,)), ((), ())),
                            preferred_element_type=jnp.float32)
    g = jax.lax.dot_general(u, wglin_ref[...],
                            dimension_numbers=(((1,), (1,)), ((), ())),
                            preferred_element_type=jnp.float32)
    iota = jax.lax.broadcasted_iota(jnp.int32, (1, E), 1)
    m0 = jnp.max(g)
    i0 = jnp.min(jnp.where(g == m0, iota, E))
    g2 = jnp.where(iota == i0, -jnp.inf, g)
    m1 = jnp.max(g2)
    i1 = jnp.min(jnp.where(g2 == m1, iota, E))
    e1 = jnp.exp(m1 - m0)
    w0 = 1.0 / (1.0 + e1)
    w1 = e1 / (1.0 + e1)
    pltpu.make_async_copy(wexp_t_hbm.at[i0], wcat_ref.at[0:F, :],
                          esem.at[0]).start()
    pltpu.make_async_copy(wexp_t_hbm.at[i1], wcat_ref.at[F:2 * F, :],
                          esem.at[1]).start()
    lanes = jax.lax.broadcasted_iota(jnp.int32, (2 * F, 2), 0)
    cols = jax.lax.broadcasted_iota(jnp.int32, (2 * F, 2), 1)
    sel = jnp.where((lanes < F) == (cols == 0), 1.0, 0.0)
    with jax.named_scope("expert_wait"):
        pltpu.make_async_copy(wexp_t_hbm.at[0], wcat_ref.at[0:F, :],
                              esem.at[0]).wait()
        pltpu.make_async_copy(wexp_t_hbm.at[0], wcat_ref.at[F:2 * F, :],
                              esem.at[1]).wait()

    for k in range(S // EB):
        xb = stash_ref[k * EB:(k + 1) * EB, :]
        zbuf_ref[k * EB:(k + 1) * EB, :] = jax.lax.dot_general(
            xb, wcat_ref[...],
            dimension_numbers=(((1,), (1,)), ((), ())),
            preferred_element_type=jnp.float32)
    EB2 = 256
    for k in range(S // EB2):
        z = zbuf_ref[k * EB2:(k + 1) * EB2, :]
        nrm2 = jnp.dot(z * z, sel,
                       preferred_element_type=jnp.float32)
        rinv = jax.lax.rsqrt(jnp.maximum(nrm2, 1e-24))
        u0 = z[:, 0:F] * rinv[:, 0:1]
        u1 = z[:, F:2 * F] * rinv[:, 1:2]
        t0 = u0 * (1.0 + jax.lax.erf(u0 * 0.7071067811865476))
        t1 = u1 * (1.0 + jax.lax.erf(u1 * 0.7071067811865476))
        y = (0.5 * w0) * t0 + (0.5 * w1) * t1
        yt_ref[:, k * EB2:(k + 1) * EB2] = y.T
        pltpu.make_async_copy(
            yt_ref.at[:, pl.ds(k * EB2, EB2)],
            out_hbm.at[:, pl.ds(k * EB2, EB2)], osem.at[k]).start()

    with jax.named_scope("out_wait"):
        for k in range(S // 256):
            pltpu.make_async_copy(
                yt_ref.at[:, pl.ds(k * 256, 256)],
                out_hbm.at[:, pl.ds(k * 256, 256)], osem.at[k]).wait()


@jax.jit
def kernel(x, W_gate_in, W_gate_lin, W_gate_out, W_experts):
    y_t = pl.pallas_call(
        _moe_kernel,
        in_specs=[
            pl.BlockSpec(memory_space=pl.ANY),
            pl.BlockSpec(memory_space=pl.ANY),
            pl.BlockSpec(memory_space=pl.ANY),
            pl.BlockSpec(memory_space=pl.ANY),
            pl.BlockSpec(memory_space=pl.ANY),
            pl.BlockSpec(memory_space=pl.ANY),
        ],
        out_specs=pl.BlockSpec(memory_space=pl.ANY),
        out_shape=jax.ShapeDtypeStruct((F, S), jnp.float32),
        compiler_params=pltpu.CompilerParams(
            vmem_limit_bytes=100 * 1024 * 1024),
        scratch_shapes=[
            pltpu.VMEM((S, D), jnp.float32),
            pltpu.VMEM((11264, 1024), jnp.float32),
            pltpu.VMEM((2 * F, D), jnp.float32),
            pltpu.VMEM((1, S), jnp.float32),
            pltpu.VMEM((H, D), jnp.float32),
            pltpu.VMEM((E, H), jnp.float32),
            pltpu.VMEM((F, S), jnp.float32),
            pltpu.VMEM((S, 2 * F), jnp.float32),
            pltpu.SemaphoreType.DMA((NC,)),
            pltpu.SemaphoreType.DMA((3,)),
            pltpu.SemaphoreType.DMA((2,)),
            pltpu.SemaphoreType.DMA((S // 256,)),
        ],
    )(x, x, W_gate_out.reshape(1, S), W_gate_in.T, W_gate_lin.T,
      W_experts.transpose(0, 2, 1))
    return y_t.T


# speedup vs baseline: 1.0022x; 1.0022x over previous
import jax
import jax.numpy as jnp
from jax.experimental import pallas as pl
from jax.experimental.pallas import tpu as pltpu

S, D, H, E, TOPK, F = 2048, 1024, 64, 16, 2, 64

CH_OFF = (0, 768, 1536, 1792)
CH_LEN = (768, 768, 256, 256)
NC = len(CH_OFF)
EB = 1024
CS = None


def _moe_kernel(x_hbm, x2_hbm, wout_hbm, wgin_t_hbm, wglin_t_hbm, wexp_t_hbm,
                out_hbm, stash_ref, pad_ref, wcat_ref, wout_ref, wgin_ref,
                wglin_ref, yt_ref, zbuf_ref, xsem, wsem, esem, osem):
    for k in range(NC):
        src_ref = x_hbm if k % 2 == 0 else x2_hbm
        pltpu.make_async_copy(
            src_ref.at[pl.ds(CH_OFF[k], CH_LEN[k]), :],
            stash_ref.at[pl.ds(CH_OFF[k], CH_LEN[k]), :], xsem.at[k]).start()
    pltpu.make_async_copy(wout_hbm, wout_ref, wsem.at[0]).start()
    pltpu.make_async_copy(wgin_t_hbm, wgin_ref, wsem.at[1]).start()
    pltpu.make_async_copy(wglin_t_hbm, wglin_ref, wsem.at[2]).start()

    with jax.named_scope("stream_gate"):
        pltpu.make_async_copy(wout_hbm, wout_ref, wsem.at[0]).wait()
        v = jnp.zeros((1, D), dtype=jnp.float32)
        for k in range(NC):
            pltpu.make_async_copy(
                x_hbm.at[pl.ds(CH_OFF[k], CH_LEN[k]), :],
                stash_ref.at[pl.ds(CH_OFF[k], CH_LEN[k]), :], xsem.at[k]).wait()
            v = v + jnp.dot(wout_ref[:, CH_OFF[k]:CH_OFF[k] + CH_LEN[k]],
                            stash_ref[CH_OFF[k]:CH_OFF[k] + CH_LEN[k], :],
                            preferred_element_type=jnp.float32)

    pltpu.make_async_copy(wgin_t_hbm, wgin_ref, wsem.at[1]).wait()
    pltpu.make_async_copy(wglin_t_hbm, wglin_ref, wsem.at[2]).wait()
    u = jax.lax.dot_general(v, wgin_ref[...],
                            dimension_numbers=(((1,), (1,)), ((), ())),
                            preferred_element_type=jnp.float32)
    g = jax.lax.dot_general(u, wglin_ref[...],
                            dimension_numbers=(((1,), (1,)), ((), ())),
                            preferred_element_type=jnp.float32)
    iota = jax.lax.broadcasted_iota(jnp.int32, (1, E), 1)
    m0 = jnp.max(g)
    i0 = jnp.min(jnp.where(g == m0, iota, E))
    g2 = jnp.where(iota == i0, -jnp.inf, g)
    m1 = jnp.max(g2)
    i1 = jnp.min(jnp.where(g2 == m1, iota, E))
    e1 = jnp.exp(m1 - m0)
    w0 = 1.0 / (1.0 + e1)
    w1 = e1 / (1.0 + e1)
    pltpu.make_async_copy(wexp_t_hbm.at[i0], wcat_ref.at[0:F, :],
                          esem.at[0]).start()
    pltpu.make_async_copy(wexp_t_hbm.at[i1], wcat_ref.at[F:2 * F, :],
                          esem.at[1]).start()
    lanes = jax.lax.broadcasted_iota(jnp.int32, (2 * F, 2), 0)
    cols = jax.lax.broadcasted_iota(jnp.int32, (2 * F, 2), 1)
    sel = jnp.where((lanes < F) == (cols == 0), 1.0, 0.0)
    with jax.named_scope("expert_wait"):
        pltpu.make_async_copy(wexp_t_hbm.at[0], wcat_ref.at[0:F, :],
                              esem.at[0]).wait()
        pltpu.make_async_copy(wexp_t_hbm.at[0], wcat_ref.at[F:2 * F, :],
                              esem.at[1]).wait()

    def mm(k):
        xb = stash_ref[k * EB:(k + 1) * EB, :]
        zbuf_ref[k * EB:(k + 1) * EB, :] = jax.lax.dot_general(
            xb, wcat_ref[...],
            dimension_numbers=(((1,), (1,)), ((), ())),
            preferred_element_type=jnp.float32)

    mm(0)
    mm(1)
    EB2 = 256
    for k in range(S // EB2):
        if (k + 1) * EB2 % EB == 0 and ((k + 1) * EB2 // EB) + 1 < S // EB + 1:
            nxt = ((k + 1) * EB2 // EB) + 1
            if nxt < S // EB:
                mm(nxt)
        z = zbuf_ref[k * EB2:(k + 1) * EB2, :]
        nrm2 = jnp.dot(z * z, sel,
                       preferred_element_type=jnp.float32)
        rinv = jax.lax.rsqrt(jnp.maximum(nrm2, 1e-24))
        u0 = z[:, 0:F] * rinv[:, 0:1]
        u1 = z[:, F:2 * F] * rinv[:, 1:2]
        t0 = u0 * (1.0 + jax.lax.erf(u0 * 0.7071067811865476))
        t1 = u1 * (1.0 + jax.lax.erf(u1 * 0.7071067811865476))
        y = (0.5 * w0) * t0 + (0.5 * w1) * t1
        yt_ref[:, k * EB2:(k + 1) * EB2] = y.T
        pltpu.make_async_copy(
            yt_ref.at[:, pl.ds(k * EB2, EB2)],
            out_hbm.at[:, pl.ds(k * EB2, EB2)], osem.at[k]).start()

    with jax.named_scope("out_wait"):
        for k in range(S // 256):
            pltpu.make_async_copy(
                yt_ref.at[:, pl.ds(k * 256, 256)],
                out_hbm.at[:, pl.ds(k * 256, 256)], osem.at[k]).wait()


@jax.jit
def kernel(x, W_gate_in, W_gate_lin, W_gate_out, W_experts):
    y_t = pl.pallas_call(
        _moe_kernel,
        in_specs=[
            pl.BlockSpec(memory_space=pl.ANY),
            pl.BlockSpec(memory_space=pl.ANY),
            pl.BlockSpec(memory_space=pl.ANY),
            pl.BlockSpec(memory_space=pl.ANY),
            pl.BlockSpec(memory_space=pl.ANY),
            pl.BlockSpec(memory_space=pl.ANY),
        ],
        out_specs=pl.BlockSpec(memory_space=pl.ANY),
        out_shape=jax.ShapeDtypeStruct((F, S), jnp.float32),
        compiler_params=pltpu.CompilerParams(
            vmem_limit_bytes=100 * 1024 * 1024),
        scratch_shapes=[
            pltpu.VMEM((S, D), jnp.float32),
            pltpu.VMEM((11264, 1024), jnp.float32),
            pltpu.VMEM((2 * F, D), jnp.float32),
            pltpu.VMEM((1, S), jnp.float32),
            pltpu.VMEM((H, D), jnp.float32),
            pltpu.VMEM((E, H), jnp.float32),
            pltpu.VMEM((F, S), jnp.float32),
            pltpu.VMEM((S, 2 * F), jnp.float32),
            pltpu.SemaphoreType.DMA((NC,)),
            pltpu.SemaphoreType.DMA((3,)),
            pltpu.SemaphoreType.DMA((2,)),
            pltpu.SemaphoreType.DMA((S // 256,)),
        ],
    )(x, x, W_gate_out.reshape(1, S), W_gate_in.T, W_gate_lin.T,
      W_experts.transpose(0, 2, 1))
    return y_t.T
